# Initial kernel scaffold; baseline (speedup 1.0000x reference)
import jax
import jax.numpy as jnp
from jax.experimental import pallas as pl


def kernel(inputs, router_w, w1, b1, ln_g, ln_b, w2, b2):
    raise NotImplementedError("write your pallas kernel here")


# speedup vs baseline: 1.1991x; 1.1991x over previous
import jax
import jax.numpy as jnp
from jax.experimental import pallas as pl


def _router_body(x_ref, w_ref, logits_ref, pt_ref):
    x = x_ref[0]
    w = w_ref[...]
    logits_ref[0] = jnp.dot(x, w, preferred_element_type=jnp.float32)
    lg_t = jax.lax.dot_general(w, x, (((0,), (1,)), ((), ())),
                               preferred_element_type=jnp.float32)
    m = jnp.max(lg_t, axis=0, keepdims=True)
    ex = jnp.exp(lg_t - m)
    pt_ref[...] = ex / jnp.sum(ex, axis=0, keepdims=True)


def _make_rank_body(S, K, CH):
    def _rank_body(prow_ref, pcol_ref, rank_ref, wd_ref):
        prow = prow_ref[0]
        irow = jax.lax.broadcasted_iota(jnp.int32, (1, S), 1)

        def chunk(c, carry):
            pc = pcol_ref[0, pl.ds(c * CH, CH), :]
            ic = c * CH + jax.lax.broadcasted_iota(jnp.int32, (CH, 1), 0)
            gt = prow > pc
            eq = (prow == pc) & (irow < ic)
            r = jnp.sum((gt | eq).astype(jnp.int32), axis=1, keepdims=True)
            rank_ref[0, pl.ds(c * CH, CH), :] = r
            return carry

        jax.lax.fori_loop(0, S // CH, chunk, 0)
        rk = rank_ref[0]
        wd_ref[0] = jnp.where(rk < K, pcol_ref[0], 0.0)

    return _rank_body


def _make_moe_body(B, S, D, E, F, K):
    bf16 = jnp.bfloat16
    f32 = jnp.float32

    def _moe_body(x_ref, rs_ref, rd_ref, wd_ref, w1_ref, b1_ref, g_ref,
                  bb_ref, w2_ref, b2_ref, out_ref):
        e = pl.program_id(0)
        jj = jax.lax.broadcasted_iota(jnp.int32, (K, 1), 0)
        G = jnp.concatenate(
            [(rs_ref[b] == jj).astype(bf16) for b in range(B)], axis=0)
        x2 = jnp.dot(G, x_ref[0], preferred_element_type=f32)
        h = jnp.dot(x2.astype(bf16), w1_ref[0],
                    preferred_element_type=f32) + b1_ref[0]
        h = 0.5 * h * (1.0 + jax.lax.erf(h * (2.0 ** -0.5)))
        mu = jnp.mean(h, axis=-1, keepdims=True)
        var = jnp.mean((h - mu) ** 2, axis=-1, keepdims=True)
        h = (h - mu) / jnp.sqrt(var + 1e-5) * g_ref[0] + bb_ref[0]
        y = jnp.dot(h.astype(bf16), w2_ref[0],
                    preferred_element_type=f32) + b2_ref[0]
        y16 = y.astype(bf16)
        jr = jax.lax.broadcasted_iota(jnp.int32, (1, K), 1)
        for b in range(B):
            c = (rd_ref[b] == jr).astype(bf16)
            contrib = jnp.dot(c, y16[b * K:(b + 1) * K],
                              preferred_element_type=f32) * wd_ref[b]

            @pl.when(e == 0)
            def _(contrib=contrib, b=b):
                out_ref[b] = contrib

            @pl.when(e != 0)
            def _(contrib=contrib, b=b):
                out_ref[b] += contrib

    return _moe_body


def kernel(inputs, router_w, w1, b1, ln_g, ln_b, w2, b2):
    B, S, D = inputs.shape
    E = router_w.shape[1]
    F = w1.shape[2]
    K = max(1, S // E)
    CH = 256
    f32 = jnp.float32
    bf16 = jnp.bfloat16

    logits, pt = pl.pallas_call(
        _router_body,
        grid=(B,),
        in_specs=[pl.BlockSpec((1, S, D), lambda b: (b, 0, 0)),
                  pl.BlockSpec((D, E), lambda b: (0, 0))],
        out_specs=[pl.BlockSpec((1, S, E), lambda b: (b, 0, 0)),
                   pl.BlockSpec((E, S), lambda b: (b, 0))],
        out_shape=[jax.ShapeDtypeStruct((B, S, E), f32),
                   jax.ShapeDtypeStruct((B * E, S), f32)],
    )(inputs, router_w)

    pt_row = pt.reshape(B * E, 1, S)
    pt_col = pt.reshape(B * E, S, 1)

    rank_col, wd_col = pl.pallas_call(
        _make_rank_body(S, K, CH),
        grid=(B * E,),
        in_specs=[pl.BlockSpec((1, 1, S), lambda i: (i, 0, 0)),
                  pl.BlockSpec((1, S, 1), lambda i: (i, 0, 0))],
        out_specs=[pl.BlockSpec((1, S, 1), lambda i: (i, 0, 0)),
                   pl.BlockSpec((1, S, 1), lambda i: (i, 0, 0))],
        out_shape=[jax.ShapeDtypeStruct((B * E, S, 1), jnp.int32),
                   jax.ShapeDtypeStruct((B * E, S, 1), f32)],
    )(pt_row, pt_col)

    rank_row = rank_col.reshape(B * E, 1, S)

    x16 = inputs.astype(bf16)
    w1b = w1.astype(bf16)
    w2b = w2.astype(bf16)
    b1r = b1.reshape(E, 1, F)
    gr = ln_g.reshape(E, 1, F)
    bbr = ln_b.reshape(E, 1, F)
    b2r = b2.reshape(E, 1, D)

    results = pl.pallas_call(
        _make_moe_body(B, S, D, E, F, K),
        grid=(E,),
        in_specs=[
            pl.BlockSpec((1, S, D), lambda e: ((B * e) // E, 0, 0)),
            pl.BlockSpec((B, 1, S), lambda e: (e, 0, 0)),
            pl.BlockSpec((B, S, 1), lambda e: (e, 0, 0)),
            pl.BlockSpec((B, S, 1), lambda e: (e, 0, 0)),
            pl.BlockSpec((1, D, F), lambda e: (e, 0, 0)),
            pl.BlockSpec((1, 1, F), lambda e: (e, 0, 0)),
            pl.BlockSpec((1, 1, F), lambda e: (e, 0, 0)),
            pl.BlockSpec((1, 1, F), lambda e: (e, 0, 0)),
            pl.BlockSpec((1, F, D), lambda e: (e, 0, 0)),
            pl.BlockSpec((1, 1, D), lambda e: (e, 0, 0)),
        ],
        out_specs=pl.BlockSpec((B, S, D), lambda e: (0, 0, 0)),
        out_shape=jax.ShapeDtypeStruct((B, S, D), f32),
    )(x16, rank_row,
      _dst_perm(rank_col, B, E), _dst_perm(wd_col, B, E),
      w1b, b1r, gr, bbr, w2b, b2r)

    return (results, logits)


def _dst_perm(a, B, E):
    return a.reshape(B, E, *a.shape[1:]).swapaxes(0, 1).reshape(a.shape)


# speedup vs baseline: 1.4079x; 1.1741x over previous
import jax
import jax.numpy as jnp
from jax.experimental import pallas as pl


def _router_body(x_ref, w_ref, logits_ref, pt_ref):
    x = x_ref[0]
    w = w_ref[...]
    logits_ref[0] = jnp.dot(x, w, preferred_element_type=jnp.float32)
    lg_t = jax.lax.dot_general(w, x, (((0,), (1,)), ((), ())),
                               preferred_element_type=jnp.float32)
    m = jnp.max(lg_t, axis=0, keepdims=True)
    ex = jnp.exp(lg_t - m)
    pt_ref[...] = ex / jnp.sum(ex, axis=0, keepdims=True)


def _make_rank_body(S, K, CH):
    def _rank_body(prow_ref, pcol_ref, rank_ref, wd_ref):
        prow = prow_ref[0]
        irow = jax.lax.broadcasted_iota(jnp.int32, (1, S), 1)

        def chunk(c, carry):
            pc = pcol_ref[0, pl.ds(c * CH, CH), :]
            ic = c * CH + jax.lax.broadcasted_iota(jnp.int32, (CH, 1), 0)
            gt = prow > pc
            eq = (prow == pc) & (irow < ic)
            r = jnp.sum((gt | eq).astype(jnp.int32), axis=1, keepdims=True)
            rank_ref[0, pl.ds(c * CH, CH), :] = r
            return carry

        jax.lax.fori_loop(0, S // CH, chunk, 0)
        rk = rank_ref[0]
        wd_ref[0] = jnp.where(rk < K, pcol_ref[0], 0.0)

    return _rank_body


def _make_ffn_in_body(B, S, D, E, F, K):
    bf16 = jnp.bfloat16
    f32 = jnp.float32

    def _ffn_in_body(x_ref, rs_ref, w1_ref, b1_ref, g_ref, bb_ref, hn_ref):
        jj = jax.lax.broadcasted_iota(jnp.int32, (K, 1), 0)
        G = jnp.concatenate(
            [(rs_ref[b] == jj).astype(bf16) for b in range(B)], axis=0)
        x2 = jnp.dot(G, x_ref[0], preferred_element_type=f32)
        w1b = w1_ref[0].astype(bf16)
        h = jnp.dot(x2.astype(bf16), w1b,
                    preferred_element_type=f32) + b1_ref[0]
        h = 0.5 * h * (1.0 + jax.lax.erf(h * (2.0 ** -0.5)))
        mu = jnp.mean(h, axis=-1, keepdims=True)
        var = jnp.mean((h - mu) ** 2, axis=-1, keepdims=True)
        h = (h - mu) / jnp.sqrt(var + 1e-5) * g_ref[0] + bb_ref[0]
        hn_ref[0] = h.astype(bf16)

    return _ffn_in_body


def _make_combine_body(B, S, D, E, F, K):
    bf16 = jnp.bfloat16
    f32 = jnp.float32

    def _combine_body(hn_ref, rd_ref, wd_ref, w2_ref, b2_ref, out_ref):
        e = pl.program_id(0)
        w2b = w2_ref[0].astype(bf16)
        y = jnp.dot(hn_ref[0], w2b,
                    preferred_element_type=f32) + b2_ref[0]
        y16 = y.astype(bf16)
        jr = jax.lax.broadcasted_iota(jnp.int32, (1, K), 1)
        for b in range(B):
            c = (rd_ref[b] == jr).astype(bf16)
            contrib = jnp.dot(c, y16[b * K:(b + 1) * K],
                              preferred_element_type=f32) * wd_ref[b]

            @pl.when(e == 0)
            def _(contrib=contrib, b=b):
                out_ref[b] = contrib

            @pl.when(e != 0)
            def _(contrib=contrib, b=b):
                out_ref[b] += contrib

    return _combine_body


def kernel(inputs, router_w, w1, b1, ln_g, ln_b, w2, b2):
    B, S, D = inputs.shape
    E = router_w.shape[1]
    F = w1.shape[2]
    K = max(1, S // E)
    CH = 256
    f32 = jnp.float32
    bf16 = jnp.bfloat16

    logits, pt = pl.pallas_call(
        _router_body,
        grid=(B,),
        in_specs=[pl.BlockSpec((1, S, D), lambda b: (b, 0, 0)),
                  pl.BlockSpec((D, E), lambda b: (0, 0))],
        out_specs=[pl.BlockSpec((1, S, E), lambda b: (b, 0, 0)),
                   pl.BlockSpec((E, S), lambda b: (b, 0))],
        out_shape=[jax.ShapeDtypeStruct((B, S, E), f32),
                   jax.ShapeDtypeStruct((B * E, S), f32)],
    )(inputs, router_w)

    pt_row = pt.reshape(B * E, 1, S)
    pt_col = pt.reshape(B * E, S, 1)

    rank_col, wd_col = pl.pallas_call(
        _make_rank_body(S, K, CH),
        grid=(B * E,),
        in_specs=[pl.BlockSpec((1, 1, S), lambda i: (i, 0, 0)),
                  pl.BlockSpec((1, S, 1), lambda i: (i, 0, 0))],
        out_specs=[pl.BlockSpec((1, S, 1), lambda i: (i, 0, 0)),
                   pl.BlockSpec((1, S, 1), lambda i: (i, 0, 0))],
        out_shape=[jax.ShapeDtypeStruct((B * E, S, 1), jnp.int32),
                   jax.ShapeDtypeStruct((B * E, S, 1), f32)],
    )(pt_row, pt_col)

    rank_row = rank_col.reshape(B * E, 1, S)

    x16 = inputs.astype(bf16)
    b1r = b1.reshape(E, 1, F)
    gr = ln_g.reshape(E, 1, F)
    bbr = ln_b.reshape(E, 1, F)
    b2r = b2.reshape(E, 1, D)

    hn = pl.pallas_call(
        _make_ffn_in_body(B, S, D, E, F, K),
        grid=(E,),
        in_specs=[
            pl.BlockSpec((1, S, D), lambda e: ((B * e) // E, 0, 0)),
            pl.BlockSpec((B, 1, S), lambda e: (e, 0, 0)),
            pl.BlockSpec((1, D, F), lambda e: (e, 0, 0)),
            pl.BlockSpec((1, 1, F), lambda e: (e, 0, 0)),
            pl.BlockSpec((1, 1, F), lambda e: (e, 0, 0)),
            pl.BlockSpec((1, 1, F), lambda e: (e, 0, 0)),
        ],
        out_specs=pl.BlockSpec((1, B * K, F), lambda e: (e, 0, 0)),
        out_shape=jax.ShapeDtypeStruct((E, B * K, F), bf16),
    )(x16, rank_row, w1, b1r, gr, bbr)

    results = pl.pallas_call(
        _make_combine_body(B, S, D, E, F, K),
        grid=(E,),
        in_specs=[
            pl.BlockSpec((1, B * K, F), lambda e: (e, 0, 0)),
            pl.BlockSpec((B, S, 1), lambda e: (e, 0, 0)),
            pl.BlockSpec((B, S, 1), lambda e: (e, 0, 0)),
            pl.BlockSpec((1, F, D), lambda e: (e, 0, 0)),
            pl.BlockSpec((1, 1, D), lambda e: (e, 0, 0)),
        ],
        out_specs=pl.BlockSpec((B, S, D), lambda e: (0, 0, 0)),
        out_shape=jax.ShapeDtypeStruct((B, S, D), f32),
    )(hn, _dst_perm(rank_col, B, E), _dst_perm(wd_col, B, E), w2, b2r)

    return (results, logits)


def _dst_perm(a, B, E):
    return a.reshape(B, E, *a.shape[1:]).swapaxes(0, 1).reshape(a.shape)


# speedup vs baseline: 1.7126x; 1.2164x over previous
import jax
import jax.numpy as jnp
from jax.experimental import pallas as pl


def _router_body(x_ref, w_ref, logits_ref, pt_ref):
    x = x_ref[0]
    w = w_ref[...]
    logits_ref[0] = jnp.dot(x, w, preferred_element_type=jnp.float32)
    lg_t = jax.lax.dot_general(w, x, (((0,), (1,)), ((), ())),
                               preferred_element_type=jnp.float32)
    m = jnp.max(lg_t, axis=0, keepdims=True)
    ex = jnp.exp(lg_t - m)
    pt_ref[...] = ex / jnp.sum(ex, axis=0, keepdims=True)


def _make_rank_body(S, K, CH):
    NB = S // CH
    f32 = jnp.float32

    def _rank_body(prow_ref, pcol_ref, rank_ref, wd_ref):
        prow = prow_ref[0]
        tri = (jax.lax.broadcasted_iota(jnp.int32, (CH, CH), 1)
               < jax.lax.broadcasted_iota(jnp.int32, (CH, CH), 0))
        for ci in range(NB):
            pc = pcol_ref[0, ci * CH:(ci + 1) * CH, :]
            acc = jnp.zeros((CH, CH), f32)
            for cj in range(NB):
                blk = prow[:, cj * CH:(cj + 1) * CH]
                if cj < ci:
                    m = blk >= pc
                elif cj > ci:
                    m = blk > pc
                else:
                    m = (blk > pc) | ((blk == pc) & tri)
                acc = acc + m.astype(f32)
            r = jnp.sum(acc, axis=1, keepdims=True).astype(jnp.int32)
            rank_ref[0, ci * CH:(ci + 1) * CH, :] = r
        rk = rank_ref[0]
        wd_ref[0] = jnp.where(rk < K, pcol_ref[0], 0.0)

    return _rank_body


def _make_ffn_in_body(B, S, D, E, F, K):
    bf16 = jnp.bfloat16
    f32 = jnp.float32

    def _ffn_in_body(x_ref, rs_ref, w1_ref, b1_ref, g_ref, bb_ref, hn_ref):
        jj = jax.lax.broadcasted_iota(jnp.int32, (K, 1), 0)
        G = jnp.concatenate(
            [(rs_ref[b] == jj).astype(bf16) for b in range(B)], axis=0)
        x2 = jnp.dot(G, x_ref[0], preferred_element_type=f32)
        w1b = w1_ref[0].astype(bf16)
        h = jnp.dot(x2.astype(bf16), w1b,
                    preferred_element_type=f32) + b1_ref[0]
        h = 0.5 * h * (1.0 + jax.lax.erf(h * (2.0 ** -0.5)))
        mu = jnp.mean(h, axis=-1, keepdims=True)
        var = jnp.mean((h - mu) ** 2, axis=-1, keepdims=True)
        h = (h - mu) / jnp.sqrt(var + 1e-5) * g_ref[0] + bb_ref[0]
        hn_ref[0] = h.astype(bf16)

    return _ffn_in_body


def _make_combine_body(B, S, D, E, F, K):
    bf16 = jnp.bfloat16
    f32 = jnp.float32

    def _combine_body(hn_ref, rd_ref, wd_ref, w2_ref, b2_ref, out_ref):
        e = pl.program_id(0)
        w2b = w2_ref[0].astype(bf16)
        y = jnp.dot(hn_ref[0], w2b,
                    preferred_element_type=f32) + b2_ref[0]
        y16 = y.astype(bf16)
        jr = jax.lax.broadcasted_iota(jnp.int32, (1, K), 1)
        for b in range(B):
            c = (rd_ref[b] == jr).astype(bf16)
            contrib = jnp.dot(c, y16[b * K:(b + 1) * K],
                              preferred_element_type=f32) * wd_ref[b]

            @pl.when(e == 0)
            def _(contrib=contrib, b=b):
                out_ref[b] = contrib

            @pl.when(e != 0)
            def _(contrib=contrib, b=b):
                out_ref[b] += contrib

    return _combine_body


def kernel(inputs, router_w, w1, b1, ln_g, ln_b, w2, b2):
    B, S, D = inputs.shape
    E = router_w.shape[1]
    F = w1.shape[2]
    K = max(1, S // E)
    CH = 256
    f32 = jnp.float32
    bf16 = jnp.bfloat16

    logits, pt = pl.pallas_call(
        _router_body,
        grid=(B,),
        in_specs=[pl.BlockSpec((1, S, D), lambda b: (b, 0, 0)),
                  pl.BlockSpec((D, E), lambda b: (0, 0))],
        out_specs=[pl.BlockSpec((1, S, E), lambda b: (b, 0, 0)),
                   pl.BlockSpec((E, S), lambda b: (b, 0))],
        out_shape=[jax.ShapeDtypeStruct((B, S, E), f32),
                   jax.ShapeDtypeStruct((B * E, S), f32)],
    )(inputs, router_w)

    pt_row = pt.reshape(B * E, 1, S)
    pt_col = pt.reshape(B * E, S, 1)

    rank_col, wd_col = pl.pallas_call(
        _make_rank_body(S, K, CH),
        grid=(B * E,),
        in_specs=[pl.BlockSpec((1, 1, S), lambda i: (i, 0, 0)),
                  pl.BlockSpec((1, S, 1), lambda i: (i, 0, 0))],
        out_specs=[pl.BlockSpec((1, S, 1), lambda i: (i, 0, 0)),
                   pl.BlockSpec((1, S, 1), lambda i: (i, 0, 0))],
        out_shape=[jax.ShapeDtypeStruct((B * E, S, 1), jnp.int32),
                   jax.ShapeDtypeStruct((B * E, S, 1), f32)],
    )(pt_row, pt_col)

    rank_row = rank_col.reshape(B * E, 1, S)

    x16 = inputs.astype(bf16)
    b1r = b1.reshape(E, 1, F)
    gr = ln_g.reshape(E, 1, F)
    bbr = ln_b.reshape(E, 1, F)
    b2r = b2.reshape(E, 1, D)

    hn = pl.pallas_call(
        _make_ffn_in_body(B, S, D, E, F, K),
        grid=(E,),
        in_specs=[
            pl.BlockSpec((1, S, D), lambda e: ((B * e) // E, 0, 0)),
            pl.BlockSpec((B, 1, S), lambda e: (e, 0, 0)),
            pl.BlockSpec((1, D, F), lambda e: (e, 0, 0)),
            pl.BlockSpec((1, 1, F), lambda e: (e, 0, 0)),
            pl.BlockSpec((1, 1, F), lambda e: (e, 0, 0)),
            pl.BlockSpec((1, 1, F), lambda e: (e, 0, 0)),
        ],
        out_specs=pl.BlockSpec((1, B * K, F), lambda e: (e, 0, 0)),
        out_shape=jax.ShapeDtypeStruct((E, B * K, F), bf16),
    )(x16, rank_row, w1, b1r, gr, bbr)

    results = pl.pallas_call(
        _make_combine_body(B, S, D, E, F, K),
        grid=(E,),
        in_specs=[
            pl.BlockSpec((1, B * K, F), lambda e: (e, 0, 0)),
            pl.BlockSpec((B, S, 1), lambda e: (e, 0, 0)),
            pl.BlockSpec((B, S, 1), lambda e: (e, 0, 0)),
            pl.BlockSpec((1, F, D), lambda e: (e, 0, 0)),
            pl.BlockSpec((1, 1, D), lambda e: (e, 0, 0)),
        ],
        out_specs=pl.BlockSpec((B, S, D), lambda e: (0, 0, 0)),
        out_shape=jax.ShapeDtypeStruct((B, S, D), f32),
    )(hn, _dst_perm(rank_col, B, E), _dst_perm(wd_col, B, E), w2, b2r)

    return (results, logits)


def _dst_perm(a, B, E):
    return a.reshape(B, E, *a.shape[1:]).swapaxes(0, 1).reshape(a.shape)
